# baseline (speedup 1.0000x reference)
import functools

import jax
import jax.numpy as jnp
from jax import lax
from jax.experimental import pallas as pl
from jax.experimental.pallas import tpu as pltpu
from jax.experimental.pallas import tpu_sc as plsc

NCODES = 8192
CDIM = 64
TOK = 8192
HALF = TOK // 2
TM = 1024
CB = 1024
NW = 32
TPW = HALF // NW


def _argmin_body(z_ref, w_ref, idx_ref, loss_ref, wsq_ref, acc_ref):
    i = pl.program_id(0)

    @pl.when(i == 0)
    def _precompute():
        acc_ref[...] = jnp.zeros((1, 1), jnp.float32)

        def wstep(c, _):
            wc = w_ref[pl.ds(c * CB, CB), :]
            wsq_ref[pl.ds(c, 1), :] = jnp.sum(wc * wc, axis=1)[None, :]
            return 0

        lax.fori_loop(0, NCODES // CB, wstep, 0)

    z = z_ref[...]
    zsq = jnp.sum(z * z, axis=1, keepdims=True)
    z2 = z + z
    ids = lax.broadcasted_iota(jnp.int32, (TM, CB), 1).astype(jnp.float32)

    bval = jnp.full((TM, 1), jnp.inf, jnp.float32)
    bidx = jnp.zeros((TM, 1), jnp.float32)
    for c in range(NCODES // CB):
        wc = w_ref[c * CB:(c + 1) * CB, :]
        p2 = lax.dot_general(z2, wc, (((1,), (1,)), ((), ())),
                             preferred_element_type=jnp.float32)
        dist = (zsq + wsq_ref[c:c + 1, :]) - p2
        lmin = jnp.min(dist, axis=1, keepdims=True)
        lidx = jnp.min(jnp.where(dist == lmin, ids, float(2 * CB)),
                       axis=1, keepdims=True)
        lidx = lidx + float(c * CB)
        upd = lmin < bval
        bval = jnp.where(upd, lmin, bval)
        bidx = jnp.where(upd, lidx, bidx)

    idx_ref[...] = bidx.astype(jnp.int32)

    acc_ref[...] += jnp.sum(bval).reshape(1, 1)

    @pl.when(i == pl.num_programs(0) - 1)
    def _finish():
        loss_ref[...] = 0.25 * acc_ref[...] / float(TOK * CDIM)


def _make_argmin_call(n_tok):
    return pl.pallas_call(
        _argmin_body,
        grid=(n_tok // TM,),
        in_specs=[
            pl.BlockSpec((TM, CDIM), lambda i: (i, 0)),
            pl.BlockSpec((NCODES, CDIM), lambda i: (0, 0)),
        ],
        out_specs=[
            pl.BlockSpec((TM, 1), lambda i: (i, 0)),
            pl.BlockSpec((1, 1), lambda i: (0, 0)),
        ],
        out_shape=[
            jax.ShapeDtypeStruct((n_tok, 1), jnp.int32),
            jax.ShapeDtypeStruct((1, 1), jnp.float32),
        ],
        scratch_shapes=[
            pltpu.VMEM((NCODES // CB, CB), jnp.float32),
            pltpu.VMEM((1, 1), jnp.float32),
        ],
    )


def _sc_gather_hist(idx_hbm, w_hbm, zeros_hbm, zq_hbm, cnt_hbm,
                    idx_v, rows_v, ones_v, cnt_sh, sem):
    c = lax.axis_index("c")
    s = lax.axis_index("s")
    wid = s * 2 + c
    base = wid * TPW

    for t in range(8):
        ones_v[pl.ds(t * 16, 16)] = jnp.ones((16,), jnp.float32)

    pltpu.sync_copy(idx_hbm.at[wid], idx_v)

    cp0 = pltpu.async_copy(w_hbm.at[idx_v.at[0]], rows_v, sem)

    @pl.when(s == 0)
    def _init_counts():
        pltpu.sync_copy(zeros_hbm, cnt_sh)

    cp0.wait()
    pltpu.sync_copy(rows_v, zq_hbm.at[pl.ds(base, TPW)])

    plsc.subcore_barrier()
    pltpu.sync_copy(ones_v, cnt_sh.at[idx_v.at[0]], add=True)
    plsc.subcore_barrier()

    @pl.when(s == 0)
    def _write_counts():
        pltpu.sync_copy(cnt_sh, cnt_hbm.at[c])


@functools.cache
def _sc_gather_hist_call():
    mesh = plsc.VectorSubcoreMesh(core_axis_name="c", subcore_axis_name="s")
    return pl.kernel(
        _sc_gather_hist,
        mesh=mesh,
        out_type=(
            jax.ShapeDtypeStruct((HALF, 128), jnp.float32),
            jax.ShapeDtypeStruct((2, NCODES), jnp.float32),
        ),
        scratch_types=(
            pltpu.VMEM((1, 128), jnp.int32),
            pltpu.VMEM((TPW, 128), jnp.float32),
            pltpu.VMEM((128,), jnp.float32),
            pltpu.VMEM_SHARED((NCODES,), jnp.float32),
            pltpu.SemaphoreType.DMA,
        ),
    )


def _scalars_body(c1_ref, c2_ref, perp_ref):
    cnt = c1_ref[...] + c2_ref[...]
    avg = jnp.sum(cnt, axis=0, keepdims=True) / float(TOK)
    ent = -jnp.sum(avg * jnp.log(avg + 1e-10))
    perp_ref[...] = jnp.exp(ent).reshape(1, 1)


_scalars_call = pl.pallas_call(
    _scalars_body,
    out_shape=jax.ShapeDtypeStruct((1, 1), jnp.float32),
)


def kernel(z, weight):
    B, C, H, W = z.shape
    z_t = jnp.transpose(z, (0, 2, 3, 1))
    z_flat = z_t.reshape(-1, C)

    amin = _make_argmin_call(HALF)
    sc = _sc_gather_hist_call()
    zeros = jnp.zeros((NCODES,), jnp.float32)
    wpad = jnp.pad(weight, ((0, 0), (0, 128 - CDIM)))

    idx_a, loss_a = amin(z_flat[:HALF], weight)
    zq_a, cnt_a = sc(idx_a.reshape(NW, 1, 128), wpad, zeros)
    idx_b, loss_b = amin(z_flat[HALF:], weight)
    zq_b, cnt_b = sc(idx_b.reshape(NW, 1, 128), wpad, zeros)

    perp2 = _scalars_call(cnt_a, cnt_b)

    zq_flat = jnp.concatenate([zq_a[:, :CDIM], zq_b[:, :CDIM]], axis=0)
    z_q = zq_flat.reshape(B, H, W, C).transpose(0, 3, 1, 2)
    idx2d = jnp.concatenate([idx_a, idx_b], axis=0)
    return (z_q, (loss_a + loss_b)[0, 0], idx2d.reshape(B, H, W),
            perp2[0, 0])


# speedup vs baseline: 1.1804x; 1.1804x over previous
import functools

import jax
import jax.numpy as jnp
from jax import lax
from jax.experimental import pallas as pl
from jax.experimental.pallas import tpu as pltpu
from jax.experimental.pallas import tpu_sc as plsc

NCODES = 8192
CDIM = 64
TOK = 8192
TM = 2048
CB = 1024
NW = 32
TPW = TOK // NW


def _argmin_body(z_ref, w_ref, idx_ref, loss_ref, wsq_ref, acc_ref):
    i = pl.program_id(0)

    @pl.when(i == 0)
    def _precompute():
        acc_ref[...] = jnp.zeros((1, 1), jnp.float32)

        def wstep(c, _):
            wc = w_ref[pl.ds(c * CB, CB), :]
            wsq_ref[pl.ds(c, 1), :] = jnp.sum(wc * wc, axis=1)[None, :]
            return 0

        lax.fori_loop(0, NCODES // CB, wstep, 0)

    z = z_ref[...]
    zsq = jnp.sum(z * z, axis=1, keepdims=True)
    z2 = z + z
    ids = lax.broadcasted_iota(jnp.int32, (TM, CB), 1).astype(jnp.float32)

    bval = jnp.full((TM, 1), jnp.inf, jnp.float32)
    bidx = jnp.zeros((TM, 1), jnp.float32)
    for c in range(NCODES // CB):
        wc = w_ref[c * CB:(c + 1) * CB, :]
        p2 = lax.dot_general(z2, wc, (((1,), (1,)), ((), ())),
                             preferred_element_type=jnp.float32)
        dist = (zsq + wsq_ref[c:c + 1, :]) - p2
        lmin = jnp.min(dist, axis=1, keepdims=True)
        lidx = jnp.min(jnp.where(dist == lmin, ids, float(2 * CB)),
                       axis=1, keepdims=True)
        lidx = lidx + float(c * CB)
        upd = lmin < bval
        bval = jnp.where(upd, lmin, bval)
        bidx = jnp.where(upd, lidx, bidx)
    idx_ref[...] = bidx.astype(jnp.int32)

    acc_ref[...] += jnp.sum(bval).reshape(1, 1)

    @pl.when(i == pl.num_programs(0) - 1)
    def _finish():
        loss_ref[...] = 0.25 * acc_ref[...] / float(TOK * CDIM)


_argmin_call = pl.pallas_call(
    _argmin_body,
    grid=(TOK // TM,),
    in_specs=[
        pl.BlockSpec((TM, CDIM), lambda i: (i, 0)),
        pl.BlockSpec((NCODES, CDIM), lambda i: (0, 0)),
    ],
    out_specs=[
        pl.BlockSpec((TM, 1), lambda i: (i, 0)),
        pl.BlockSpec((1, 1), lambda i: (0, 0)),
    ],
    out_shape=[
        jax.ShapeDtypeStruct((TOK, 1), jnp.int32),
        jax.ShapeDtypeStruct((1, 1), jnp.float32),
    ],
    scratch_shapes=[
        pltpu.VMEM((NCODES // CB, CB), jnp.float32),
        pltpu.VMEM((1, 1), jnp.float32),
    ],
)


@functools.cache
def _sc_gather_hist_call():
    mesh = plsc.VectorSubcoreMesh(core_axis_name="c", subcore_axis_name="s")
    return pl.kernel(
        _sc_gather_hist,
        mesh=mesh,
        out_type=(
            jax.ShapeDtypeStruct((TOK, 128), jnp.float32),
            jax.ShapeDtypeStruct((2, NCODES), jnp.float32),
        ),
        scratch_types=(
            pltpu.VMEM((2, 128), jnp.int32),
            pltpu.VMEM((TPW, 128), jnp.float32),
            pltpu.VMEM((128,), jnp.float32),
            pltpu.VMEM_SHARED((NCODES,), jnp.float32),
            pltpu.SemaphoreType.DMA,
        ),
    )


def _sc_gather_hist(idx_hbm, w_hbm, zeros_hbm, zq_hbm, cnt_hbm,
                    idx_v, rows_v, ones_v, cnt_sh, sem):
    c = lax.axis_index("c")
    s = lax.axis_index("s")
    wid = s * 2 + c
    base = wid * TPW

    for t in range(8):
        ones_v[pl.ds(t * 16, 16)] = jnp.ones((16,), jnp.float32)

    pltpu.sync_copy(idx_hbm.at[wid], idx_v)

    cp0 = pltpu.async_copy(w_hbm.at[idx_v.at[0]], rows_v.at[pl.ds(0, 128)],
                           sem)
    cp1 = pltpu.async_copy(w_hbm.at[idx_v.at[1]], rows_v.at[pl.ds(128, 128)],
                           sem)

    @pl.when(s == 0)
    def _init_counts():
        pltpu.sync_copy(zeros_hbm, cnt_sh)

    cp0.wait()
    cp1.wait()
    pltpu.sync_copy(rows_v, zq_hbm.at[pl.ds(base, TPW)])

    plsc.subcore_barrier()
    pltpu.sync_copy(ones_v, cnt_sh.at[idx_v.at[0]], add=True)
    pltpu.sync_copy(ones_v, cnt_sh.at[idx_v.at[1]], add=True)
    plsc.subcore_barrier()

    @pl.when(s == 0)
    def _write_counts():
        pltpu.sync_copy(cnt_sh, cnt_hbm.at[c])


def _scalars_body(c_ref, perp_ref):
    avg = jnp.sum(c_ref[...], axis=0, keepdims=True) / float(TOK)
    ent = -jnp.sum(avg * jnp.log(avg + 1e-10))
    perp_ref[...] = jnp.exp(ent).reshape(1, 1)


_scalars_call = pl.pallas_call(
    _scalars_body,
    out_shape=jax.ShapeDtypeStruct((1, 1), jnp.float32),
)


def kernel(z, weight):
    B, C, H, W = z.shape
    z_t = jnp.transpose(z, (0, 2, 3, 1))
    z_flat = z_t.reshape(-1, C)

    idx2d, loss2 = _argmin_call(z_flat, weight)

    idx_r = idx2d.reshape(NW, 2, 128)
    zeros = jnp.zeros((NCODES,), jnp.float32)
    wpad = jnp.pad(weight, ((0, 0), (0, 128 - CDIM)))
    zq_pad, cnt = _sc_gather_hist_call()(idx_r, wpad, zeros)

    perp2 = _scalars_call(cnt)

    z_q = zq_pad[:, :CDIM].reshape(B, H, W, C).transpose(0, 3, 1, 2)
    return (z_q, loss2[0, 0], idx2d.reshape(B, H, W), perp2[0, 0])


# speedup vs baseline: 1.1949x; 1.0123x over previous
import functools

import jax
import jax.numpy as jnp
from jax import lax
from jax.experimental import pallas as pl
from jax.experimental.pallas import tpu as pltpu
from jax.experimental.pallas import tpu_sc as plsc

NCODES = 8192
CDIM = 64
TOK = 8192
TM = 1024
CB = 1024
NW = 32
TPW = TOK // NW


def _argmin_body(z_ref, w_ref, idx_ref, loss_ref, wsq_ref, acc_ref):
    i = pl.program_id(0)

    @pl.when(i == 0)
    def _precompute():
        acc_ref[...] = jnp.zeros((1, 1), jnp.float32)

        def wstep(c, _):
            wc = w_ref[pl.ds(c * CB, CB), :]
            wsq_ref[pl.ds(c, 1), :] = jnp.sum(wc * wc, axis=1)[None, :]
            return 0

        lax.fori_loop(0, NCODES // CB, wstep, 0)

    z = z_ref[...]
    zsq = jnp.sum(z * z, axis=1, keepdims=True)
    z2 = z + z
    ids = lax.broadcasted_iota(jnp.int32, (TM, CB), 1).astype(jnp.float32)

    bval = jnp.full((TM, 1), jnp.inf, jnp.float32)
    bidx = jnp.zeros((TM, 1), jnp.float32)
    for c in range(NCODES // CB):
        wc = w_ref[c * CB:(c + 1) * CB, :]
        p2 = lax.dot_general(z2, wc, (((1,), (1,)), ((), ())),
                             preferred_element_type=jnp.float32)
        dist = (zsq + wsq_ref[c:c + 1, :]) - p2
        lmin = jnp.min(dist, axis=1, keepdims=True)
        lidx = jnp.min(jnp.where(dist == lmin, ids, float(2 * CB)),
                       axis=1, keepdims=True)
        lidx = lidx + float(c * CB)
        upd = lmin < bval
        bval = jnp.where(upd, lmin, bval)
        bidx = jnp.where(upd, lidx, bidx)
    idx_ref[...] = bidx.astype(jnp.int32)

    acc_ref[...] += jnp.sum(bval).reshape(1, 1)

    @pl.when(i == pl.num_programs(0) - 1)
    def _finish():
        loss_ref[...] = 0.25 * acc_ref[...] / float(TOK * CDIM)


_argmin_call = pl.pallas_call(
    _argmin_body,
    grid=(TOK // TM,),
    in_specs=[
        pl.BlockSpec((TM, CDIM), lambda i: (i, 0)),
        pl.BlockSpec((NCODES, CDIM), lambda i: (0, 0)),
    ],
    out_specs=[
        pl.BlockSpec((TM, 1), lambda i: (i, 0)),
        pl.BlockSpec((1, 1), lambda i: (0, 0)),
    ],
    out_shape=[
        jax.ShapeDtypeStruct((TOK, 1), jnp.int32),
        jax.ShapeDtypeStruct((1, 1), jnp.float32),
    ],
    scratch_shapes=[
        pltpu.VMEM((NCODES // CB, CB), jnp.float32),
        pltpu.VMEM((1, 1), jnp.float32),
    ],
)


@functools.cache
def _sc_gather_hist_call():
    mesh = plsc.VectorSubcoreMesh(core_axis_name="c", subcore_axis_name="s")
    return pl.kernel(
        _sc_gather_hist,
        mesh=mesh,
        out_type=(
            jax.ShapeDtypeStruct((TOK, 128), jnp.float32),
            jax.ShapeDtypeStruct((2, NCODES), jnp.float32),
        ),
        scratch_types=(
            pltpu.VMEM((2, 128), jnp.int32),
            pltpu.VMEM((TPW, 128), jnp.float32),
            pltpu.VMEM((128,), jnp.float32),
            pltpu.VMEM_SHARED((NCODES,), jnp.float32),
            pltpu.SemaphoreType.DMA,
        ),
    )


def _sc_gather_hist(idx_hbm, w_hbm, zeros_hbm, zq_hbm, cnt_hbm,
                    idx_v, rows_v, ones_v, cnt_sh, sem):
    c = lax.axis_index("c")
    s = lax.axis_index("s")
    wid = s * 2 + c
    base = wid * TPW

    for t in range(8):
        ones_v[pl.ds(t * 16, 16)] = jnp.ones((16,), jnp.float32)

    pltpu.sync_copy(idx_hbm.at[wid], idx_v)

    cp0 = pltpu.async_copy(w_hbm.at[idx_v.at[0]], rows_v.at[pl.ds(0, 128)],
                           sem)
    cp1 = pltpu.async_copy(w_hbm.at[idx_v.at[1]], rows_v.at[pl.ds(128, 128)],
                           sem)

    @pl.when(s == 0)
    def _init_counts():
        pltpu.sync_copy(zeros_hbm, cnt_sh)

    cp0.wait()
    cp1.wait()
    pltpu.sync_copy(rows_v, zq_hbm.at[pl.ds(base, TPW)])

    plsc.subcore_barrier()
    pltpu.sync_copy(ones_v, cnt_sh.at[idx_v.at[0]], add=True)
    pltpu.sync_copy(ones_v, cnt_sh.at[idx_v.at[1]], add=True)
    plsc.subcore_barrier()

    @pl.when(s == 0)
    def _write_counts():
        pltpu.sync_copy(cnt_sh, cnt_hbm.at[c])


def _scalars_body(c_ref, perp_ref):
    avg = jnp.sum(c_ref[...], axis=0, keepdims=True) / float(TOK)
    ent = -jnp.sum(avg * jnp.log(avg + 1e-10))
    perp_ref[...] = jnp.exp(ent).reshape(1, 1)


_scalars_call = pl.pallas_call(
    _scalars_body,
    out_shape=jax.ShapeDtypeStruct((1, 1), jnp.float32),
)


def kernel(z, weight):
    B, C, H, W = z.shape
    z_t = jnp.transpose(z, (0, 2, 3, 1))
    z_flat = z_t.reshape(-1, C)

    idx2d, loss2 = _argmin_call(z_flat, weight)

    idx_r = idx2d.reshape(NW, 2, 128)
    zeros = jnp.zeros((NCODES,), jnp.float32)
    wpad = jnp.pad(weight, ((0, 0), (0, 128 - CDIM)))
    zq_pad, cnt = _sc_gather_hist_call()(idx_r, wpad, zeros)

    perp2 = _scalars_call(cnt)

    z_q = zq_pad[:, :CDIM].reshape(B, H, W, C).transpose(0, 3, 1, 2)
    return (z_q, loss2[0, 0], idx2d.reshape(B, H, W), perp2[0, 0])
